# Initial kernel scaffold; baseline (speedup 1.0000x reference)
import jax
import jax.numpy as jnp
from jax.experimental import pallas as pl


def kernel(x, edge_index, weight):
    raise NotImplementedError("write your pallas kernel here")


# speedup vs baseline: 19.6180x; 19.6180x over previous
import functools

import jax
import jax.numpy as jnp
from jax import lax
from jax.experimental import pallas as pl
from jax.experimental.pallas import tpu as pltpu
from jax.experimental.pallas import tpu_sc as plsc

N_NODES = 10000
N_EDGES = 320000
D = 128
DH = 64
NT = 16
RPT = 640
NPAD = NT * RPT
RB = 128
K = 128
CHUNKS = 160
CB = 8
EPAD = NT * CHUNKS * K
TRASH = N_NODES


def _sc_aggregate(x_packed, rowp, colp):
    mesh = plsc.VectorSubcoreMesh(core_axis_name="c", subcore_axis_name="s")

    @functools.partial(
        pl.kernel,
        out_type=jax.ShapeDtypeStruct((2 * NPAD, DH), jnp.float32),
        mesh=mesh,
        compiler_params=pltpu.CompilerParams(use_tc_tiling_on_sc=False),
        scratch_types=[
            pltpu.VMEM_SHARED((NPAD, DH), jnp.float32),
            pltpu.VMEM_SHARED((NPAD, DH), jnp.float32),
            pltpu.VMEM_SHARED((NPAD,), jnp.float32),
            pltpu.VMEM((CB, K), jnp.int32),
            pltpu.VMEM((CB, K), jnp.int32),
            pltpu.VMEM((K,), jnp.float32),
            pltpu.VMEM((RPT + 16,), jnp.float32),
            pltpu.VMEM((RB, DH), jnp.float32),
            pltpu.VMEM((K, DH), jnp.float32),
        ],
    )
    def body(x_hbm, rowp_hbm, colp_hbm, out_hbm,
             z_sh, acc_sh, deg_sh, col_v, row_v, ones_v, dis_v, rb_v, g_v):
        c = lax.axis_index("c")
        s = lax.axis_index("s")
        r0 = s * RPT
        h0 = c * NPAD

        one16 = jnp.ones((16,), jnp.float32)
        zero16 = jnp.zeros((16,), jnp.float32)
        for i in range(K // 16):
            ones_v[pl.ds(16 * i, 16)] = one16

        def zero_row(i, _):
            for q in range(DH // 16):
                rb_v[i, pl.ds(16 * q, 16)] = zero16
            return 0
        lax.fori_loop(0, RB, zero_row, 0)
        for b in range(RPT // RB):
            pltpu.sync_copy(rb_v, acc_sh.at[pl.ds(r0 + RB * b, RB), :])

        def zero_deg(i, _):
            dis_v[pl.ds(16 * i, 16)] = zero16
            return 0
        lax.fori_loop(0, (RPT + 16) // 16, zero_deg, 0)
        pltpu.sync_copy(dis_v.at[pl.ds(0, RPT)], deg_sh.at[pl.ds(r0, RPT)])

        plsc.subcore_barrier()

        def deg_block(bk, _):
            pltpu.sync_copy(
                colp_hbm.at[pl.ds(s * CHUNKS + CB * bk, CB), :], col_v)
            for j in range(CB):
                pltpu.sync_copy(ones_v, deg_sh.at[col_v.at[j]], add=True)
            return 0
        lax.fori_loop(0, CHUNKS // CB, deg_block, 0)

        plsc.subcore_barrier()

        pltpu.sync_copy(deg_sh.at[pl.ds(r0, RPT)], dis_v.at[pl.ds(0, RPT)])

        def rsqrt_step(i, _):
            d = jnp.maximum(dis_v[pl.ds(16 * i, 16)], 1.0)
            y = 1.0 / d
            for _ in range(22):
                y = y * (1.5 - 0.5 * d * y * y)
            dis_v[pl.ds(16 * i, 16)] = jnp.where(
                dis_v[pl.ds(16 * i, 16)] > 0.5, y, 0.0)
            return 0
        lax.fori_loop(0, RPT // 16, rsqrt_step, 0)

        def z_block(b, _):
            pltpu.sync_copy(x_hbm.at[pl.ds(h0 + r0 + RB * b, RB), :], rb_v)

            def scale_row(i, _):
                dv = dis_v[pl.ds(RB * b + i, 16)][0]
                for q in range(DH // 16):
                    sl = pl.ds(16 * q, 16)
                    rb_v[i, sl] = rb_v[i, sl] * dv
                return 0
            lax.fori_loop(0, RB, scale_row, 0)
            pltpu.sync_copy(rb_v, z_sh.at[pl.ds(r0 + RB * b, RB), :])
            return 0
        lax.fori_loop(0, RPT // RB, z_block, 0)

        plsc.subcore_barrier()

        def edge_block(bk, _):
            pltpu.sync_copy(
                colp_hbm.at[pl.ds(s * CHUNKS + CB * bk, CB), :], col_v)
            pltpu.sync_copy(
                rowp_hbm.at[pl.ds(s * CHUNKS + CB * bk, CB), :], row_v)
            for j in range(CB):
                pltpu.sync_copy(z_sh.at[col_v.at[j]], g_v)
                pltpu.sync_copy(g_v, acc_sh.at[row_v.at[j]], add=True)
            return 0
        lax.fori_loop(0, CHUNKS // CB, edge_block, 0)

        plsc.subcore_barrier()

        def out_block(b, _):
            pltpu.sync_copy(acc_sh.at[pl.ds(r0 + RB * b, RB), :], rb_v)

            def scale_row(i, _):
                dv = dis_v[pl.ds(RB * b + i, 16)][0]
                for q in range(DH // 16):
                    sl = pl.ds(16 * q, 16)
                    rb_v[i, sl] = rb_v[i, sl] * dv
                return 0
            lax.fori_loop(0, RB, scale_row, 0)
            pltpu.sync_copy(
                rb_v, out_hbm.at[pl.ds(h0 + r0 + RB * b, RB), :])
            return 0
        lax.fori_loop(0, RPT // RB, out_block, 0)

    return body(x_packed, rowp, colp)


def _matmul_packed(agg_packed, wt):
    def body(a_lo_ref, a_hi_ref, wt_lo_ref, wt_hi_ref, o_ref):
        o_ref[...] = (
            lax.dot_general(a_lo_ref[...], wt_lo_ref[...],
                            (((1,), (0,)), ((), ())),
                            preferred_element_type=jnp.float32) +
            lax.dot_general(a_hi_ref[...], wt_hi_ref[...],
                            (((1,), (0,)), ((), ())),
                            preferred_element_type=jnp.float32))

    nblk = 8
    rb = NPAD // nblk
    return pl.pallas_call(
        body,
        grid=(nblk,),
        in_specs=[
            pl.BlockSpec((rb, DH), lambda i: (i, 0)),
            pl.BlockSpec((rb, DH), lambda i: (nblk + i, 0)),
            pl.BlockSpec((DH, D), lambda i: (0, 0)),
            pl.BlockSpec((DH, D), lambda i: (1, 0)),
        ],
        out_specs=pl.BlockSpec((rb, D), lambda i: (i, 0)),
        out_shape=jax.ShapeDtypeStruct((NPAD, D), jnp.float32),
    )(agg_packed, agg_packed, wt, wt)


def kernel(x, edge_index, weight):
    x_pad = jnp.pad(x, ((0, NPAD - N_NODES), (0, 0)))
    x_packed = x_pad.reshape(NPAD, 2, DH).transpose(1, 0, 2).reshape(
        2 * NPAD, DH)
    pad = jnp.full((EPAD - N_EDGES,), TRASH, jnp.int32)
    rowp = jnp.concatenate([edge_index[0], pad]).reshape(NT * CHUNKS, K)
    colp = jnp.concatenate([edge_index[1], pad]).reshape(NT * CHUNKS, K)
    agg_packed = _sc_aggregate(x_packed, rowp, colp)
    out_pad = _matmul_packed(agg_packed, weight.T)
    return out_pad[:N_NODES]
